# Initial kernel scaffold; baseline (speedup 1.0000x reference)
import jax
import jax.numpy as jnp
from jax.experimental import pallas as pl


def kernel(x, codebook, W_in, b_in, W_out, b_out):
    raise NotImplementedError("write your pallas kernel here")


# speedup vs baseline: 1.0839x; 1.0839x over previous
import jax
import jax.numpy as jnp
from jax import lax
from jax.experimental import pallas as pl
from jax.experimental.pallas import tpu as pltpu

_MB = 256
_KC = 2048


def _fused_body(x_ref, c_ref, w_in_ref, b_in_ref, w_out_ref, b_out_ref,
                out_ref, nearest_ref):
    K = c_ref.shape[0]
    xb = x_ref[0]
    p = jnp.dot(w_in_ref[...], xb, preferred_element_type=jnp.float32)
    p = p + b_in_ref[...]

    cnorm = jnp.sum(c_ref[...] * c_ref[...], axis=1, keepdims=True)

    run_min = jnp.full((1, _MB), jnp.inf, dtype=jnp.float32)
    run_idx = jnp.zeros((1, _MB), dtype=jnp.int32)
    for k0 in range(0, K, _KC):
        c_ch = c_ref[pl.ds(k0, _KC), :]
        d = cnorm[k0:k0 + _KC, :] - 2.0 * jnp.dot(
            c_ch, p, preferred_element_type=jnp.float32)
        mnc = jnp.min(d, axis=0, keepdims=True)
        iota = lax.broadcasted_iota(jnp.int32, (_KC, _MB), 0) + k0
        idxc = jnp.min(jnp.where(d == mnc, iota, K), axis=0, keepdims=True)
        better = mnc < run_min
        run_idx = jnp.where(better, idxc, run_idx)
        run_min = jnp.minimum(run_min, mnc)

    acc = jnp.zeros((c_ref.shape[1], _MB), dtype=jnp.float32)
    for k0 in range(0, K, _KC):
        iota = lax.broadcasted_iota(jnp.int32, (_KC, _MB), 0) + k0
        onehot = (iota == run_idx).astype(jnp.float32)
        acc = acc + lax.dot_general(
            c_ref[pl.ds(k0, _KC), :], onehot,
            (((0,), (0,)), ((), ())),
            preferred_element_type=jnp.float32)

    nearest_ref[0] = acc.T
    out_t = jnp.dot(w_out_ref[...], acc, preferred_element_type=jnp.float32)
    out_ref[0] = out_t + b_out_ref[...]


def kernel(x, codebook, W_in, b_in, W_out, b_out):
    b, c, h, w = x.shape
    hw = h * w
    K, L = codebook.shape[2], codebook.shape[3]
    xf = x.reshape(b, c, hw)
    C = codebook.reshape(K, L)

    grid = (b, hw // _MB)
    out_t, nearest = pl.pallas_call(
        _fused_body,
        grid=grid,
        in_specs=[
            pl.BlockSpec((1, c, _MB), lambda i, j: (i, 0, j)),
            pl.BlockSpec((K, L), lambda i, j: (0, 0)),
            pl.BlockSpec((L, c), lambda i, j: (0, 0)),
            pl.BlockSpec((L, 1), lambda i, j: (0, 0)),
            pl.BlockSpec((c, L), lambda i, j: (0, 0)),
            pl.BlockSpec((c, 1), lambda i, j: (0, 0)),
        ],
        out_specs=[
            pl.BlockSpec((1, c, _MB), lambda i, j: (i, 0, j)),
            pl.BlockSpec((1, _MB, L), lambda i, j: (i, j, 0)),
        ],
        out_shape=[
            jax.ShapeDtypeStruct((b, c, hw), jnp.float32),
            jax.ShapeDtypeStruct((b, hw, L), jnp.float32),
        ],
        compiler_params=pltpu.CompilerParams(
            dimension_semantics=("parallel", "parallel")),
    )(xf, C, W_in, b_in.reshape(L, 1), W_out, b_out.reshape(c, 1))

    return out_t.reshape(b, c, h, w), nearest
